# baseline (speedup 1.0000x reference)
import functools

import jax
import jax.numpy as jnp
from jax import lax
from jax.experimental import pallas as pl
from jax.experimental.pallas import tpu as pltpu
from jax.experimental.pallas import tpu_sc as plsc

NUM_STEPS = 1000
BATCH = 16384
NC = 2
NS = 16
NW = NC * NS
B_PER_W = BATCH // NW
CHUNK = 128


@functools.partial(
    pl.kernel,
    mesh=plsc.VectorSubcoreMesh(core_axis_name="c", subcore_axis_name="s"),
    compiler_params=pltpu.CompilerParams(needs_layout_passes=False),
    out_type=(
        jax.ShapeDtypeStruct((BATCH,), jnp.float32),
        jax.ShapeDtypeStruct((BATCH,), jnp.float32),
    ),
    scratch_types=[
        pltpu.VMEM((B_PER_W,), jnp.int32),
        pltpu.VMEM((B_PER_W,), jnp.float32),
        pltpu.VMEM((B_PER_W,), jnp.float32),
        pltpu.SemaphoreType.DMA,
        pltpu.SemaphoreType.DMA,
    ],
)
def _gather_sc(t_hbm, beta_hbm, alpha_hbm, alpha_out, beta_out,
               idx_v, oa_v, ob_v, sem_a, sem_b):
    wid = lax.axis_index("s") * NC + lax.axis_index("c")
    base = wid * B_PER_W
    pltpu.sync_copy(t_hbm.at[pl.ds(base, B_PER_W)], idx_v)
    gathers = []
    for j in range(B_PER_W // CHUNK):
        sl = pl.ds(j * CHUNK, CHUNK)
        cp_a = pltpu.make_async_copy(alpha_hbm.at[idx_v.at[sl]], oa_v.at[sl],
                                     sem_a)
        cp_b = pltpu.make_async_copy(beta_hbm.at[idx_v.at[sl]], ob_v.at[sl],
                                     sem_b)
        cp_a.start()
        cp_b.start()
        gathers.append((cp_a, cp_b))
    for cp_a, cp_b in gathers:
        cp_a.wait()
        cp_b.wait()
    cp_oa = pltpu.make_async_copy(oa_v, alpha_out.at[pl.ds(base, B_PER_W)],
                                  sem_a)
    cp_ob = pltpu.make_async_copy(ob_v, beta_out.at[pl.ds(base, B_PER_W)],
                                  sem_b)
    cp_oa.start()
    cp_ob.start()
    cp_oa.wait()
    cp_ob.wait()


def kernel(t, beta_t, alpha_t):
    alpha_g, beta_g = _gather_sc(t.astype(jnp.int32),
                                 beta_t.astype(jnp.float32),
                                 alpha_t.astype(jnp.float32))
    return alpha_g, beta_g


# speedup vs baseline: 1.3385x; 1.3385x over previous
import functools

import jax
import jax.numpy as jnp
from jax import lax
from jax.experimental import pallas as pl
from jax.experimental.pallas import tpu as pltpu
from jax.experimental.pallas import tpu_sc as plsc

NUM_STEPS = 1000
BATCH = 16384
NC = 2
NS = 16
NW = NC * NS
LANES = 16
B_PER_W = BATCH // NW
HALF = B_PER_W // 2


@functools.partial(
    pl.kernel,
    mesh=plsc.VectorSubcoreMesh(core_axis_name="c", subcore_axis_name="s"),
    compiler_params=pltpu.CompilerParams(needs_layout_passes=False),
    out_type=(
        jax.ShapeDtypeStruct((BATCH,), jnp.float32),
        jax.ShapeDtypeStruct((BATCH,), jnp.float32),
    ),
    scratch_types=[
        pltpu.VMEM((B_PER_W,), jnp.int32),
        pltpu.VMEM((NUM_STEPS,), jnp.float32),
        pltpu.VMEM((NUM_STEPS,), jnp.float32),
        pltpu.VMEM((B_PER_W,), jnp.float32),
        pltpu.VMEM((B_PER_W,), jnp.float32),
        pltpu.SemaphoreType.DMA,
        pltpu.SemaphoreType.DMA,
        pltpu.SemaphoreType.DMA,
        pltpu.SemaphoreType.DMA,
    ],
)
def _gather_sc(t_hbm, beta_hbm, alpha_hbm, alpha_out, beta_out,
               idx_v, beta_v, alpha_v, oa_v, ob_v,
               sem_i0, sem_i1, sem_t, sem_o):
    wid = lax.axis_index("s") * NC + lax.axis_index("c")
    base = wid * B_PER_W
    cp_i0 = pltpu.make_async_copy(t_hbm.at[pl.ds(base, HALF)],
                                  idx_v.at[pl.ds(0, HALF)], sem_i0)
    cp_i1 = pltpu.make_async_copy(t_hbm.at[pl.ds(base + HALF, HALF)],
                                  idx_v.at[pl.ds(HALF, HALF)], sem_i1)
    cp_ta = pltpu.make_async_copy(alpha_hbm, alpha_v, sem_t)
    cp_tb = pltpu.make_async_copy(beta_hbm, beta_v, sem_t)
    cp_i0.start()
    cp_ta.start()
    cp_tb.start()
    cp_i1.start()
    cp_i0.wait()
    cp_ta.wait()
    cp_tb.wait()
    for i in range(HALF // LANES):
        sl = pl.ds(i * LANES, LANES)
        idx = idx_v[sl]
        oa_v[sl] = plsc.load_gather(alpha_v, [idx])
        ob_v[sl] = plsc.load_gather(beta_v, [idx])
    cp_oa0 = pltpu.make_async_copy(oa_v.at[pl.ds(0, HALF)],
                                   alpha_out.at[pl.ds(base, HALF)], sem_o)
    cp_ob0 = pltpu.make_async_copy(ob_v.at[pl.ds(0, HALF)],
                                   beta_out.at[pl.ds(base, HALF)], sem_o)
    cp_oa0.start()
    cp_ob0.start()
    cp_i1.wait()
    for i in range(HALF // LANES, B_PER_W // LANES):
        sl = pl.ds(i * LANES, LANES)
        idx = idx_v[sl]
        oa_v[sl] = plsc.load_gather(alpha_v, [idx])
        ob_v[sl] = plsc.load_gather(beta_v, [idx])
    cp_oa1 = pltpu.make_async_copy(oa_v.at[pl.ds(HALF, HALF)],
                                   alpha_out.at[pl.ds(base + HALF, HALF)],
                                   sem_o)
    cp_ob1 = pltpu.make_async_copy(ob_v.at[pl.ds(HALF, HALF)],
                                   beta_out.at[pl.ds(base + HALF, HALF)],
                                   sem_o)
    cp_oa1.start()
    cp_ob1.start()
    cp_oa0.wait()
    cp_ob0.wait()
    cp_oa1.wait()
    cp_ob1.wait()


def kernel(t, beta_t, alpha_t):
    alpha_g, beta_g = _gather_sc(t.astype(jnp.int32),
                                 beta_t.astype(jnp.float32),
                                 alpha_t.astype(jnp.float32))
    return alpha_g, beta_g


# speedup vs baseline: 1.4409x; 1.0765x over previous
import functools

import jax
import jax.numpy as jnp
from jax import lax
from jax.experimental import pallas as pl
from jax.experimental.pallas import tpu as pltpu
from jax.experimental.pallas import tpu_sc as plsc

NUM_STEPS = 1000
BATCH = 16384
NC = 1
NS = 16
NW = NC * NS
LANES = 16
B_PER_W = BATCH // NW
HALF = B_PER_W // 2


@functools.partial(
    pl.kernel,
    mesh=plsc.VectorSubcoreMesh(core_axis_name="c", subcore_axis_name="s", num_cores=1),
    compiler_params=pltpu.CompilerParams(needs_layout_passes=False),
    out_type=(
        jax.ShapeDtypeStruct((BATCH,), jnp.float32),
        jax.ShapeDtypeStruct((BATCH,), jnp.float32),
    ),
    scratch_types=[
        pltpu.VMEM((B_PER_W,), jnp.int32),
        pltpu.VMEM((NUM_STEPS,), jnp.float32),
        pltpu.VMEM((NUM_STEPS,), jnp.float32),
        pltpu.VMEM((B_PER_W,), jnp.float32),
        pltpu.VMEM((B_PER_W,), jnp.float32),
        pltpu.SemaphoreType.DMA,
        pltpu.SemaphoreType.DMA,
        pltpu.SemaphoreType.DMA,
        pltpu.SemaphoreType.DMA,
    ],
)
def _gather_sc(t_hbm, beta_hbm, alpha_hbm, alpha_out, beta_out,
               idx_v, beta_v, alpha_v, oa_v, ob_v,
               sem_i0, sem_i1, sem_t, sem_o):
    wid = lax.axis_index("s") * NC + lax.axis_index("c")
    base = wid * B_PER_W
    cp_i0 = pltpu.make_async_copy(t_hbm.at[pl.ds(base, HALF)],
                                  idx_v.at[pl.ds(0, HALF)], sem_i0)
    cp_i1 = pltpu.make_async_copy(t_hbm.at[pl.ds(base + HALF, HALF)],
                                  idx_v.at[pl.ds(HALF, HALF)], sem_i1)
    cp_ta = pltpu.make_async_copy(alpha_hbm, alpha_v, sem_t)
    cp_tb = pltpu.make_async_copy(beta_hbm, beta_v, sem_t)
    cp_i0.start()
    cp_ta.start()
    cp_tb.start()
    cp_i1.start()
    cp_i0.wait()
    cp_ta.wait()
    cp_tb.wait()
    for i in range(HALF // LANES):
        sl = pl.ds(i * LANES, LANES)
        idx = idx_v[sl]
        oa_v[sl] = plsc.load_gather(alpha_v, [idx])
        ob_v[sl] = plsc.load_gather(beta_v, [idx])
    cp_oa0 = pltpu.make_async_copy(oa_v.at[pl.ds(0, HALF)],
                                   alpha_out.at[pl.ds(base, HALF)], sem_o)
    cp_ob0 = pltpu.make_async_copy(ob_v.at[pl.ds(0, HALF)],
                                   beta_out.at[pl.ds(base, HALF)], sem_o)
    cp_oa0.start()
    cp_ob0.start()
    cp_i1.wait()
    for i in range(HALF // LANES, B_PER_W // LANES):
        sl = pl.ds(i * LANES, LANES)
        idx = idx_v[sl]
        oa_v[sl] = plsc.load_gather(alpha_v, [idx])
        ob_v[sl] = plsc.load_gather(beta_v, [idx])
    cp_oa1 = pltpu.make_async_copy(oa_v.at[pl.ds(HALF, HALF)],
                                   alpha_out.at[pl.ds(base + HALF, HALF)],
                                   sem_o)
    cp_ob1 = pltpu.make_async_copy(ob_v.at[pl.ds(HALF, HALF)],
                                   beta_out.at[pl.ds(base + HALF, HALF)],
                                   sem_o)
    cp_oa1.start()
    cp_ob1.start()
    cp_oa0.wait()
    cp_ob0.wait()
    cp_oa1.wait()
    cp_ob1.wait()


def kernel(t, beta_t, alpha_t):
    alpha_g, beta_g = _gather_sc(t.astype(jnp.int32),
                                 beta_t.astype(jnp.float32),
                                 alpha_t.astype(jnp.float32))
    return alpha_g, beta_g
